# baseline (device time: 11075 ns/iter reference)
import jax
import jax.numpy as jnp
from jax import lax
from jax.experimental import pallas as pl
from jax.experimental.pallas import tpu as pltpu

N_DEV = 8


def kernel(x):
    m_per, n = x.shape

    m_half = m_per // 2

    def body(x_ref, out_ref, buf_ref, comm_ref, copy_sems, send_sems, recv_sems):
        my_pos = lax.axis_index("i")
        peers = [(my_pos + off) % N_DEV for off in range(1, N_DEV)]

        barrier_sem = pltpu.get_barrier_semaphore()
        for p in peers:
            pl.semaphore_signal(
                barrier_sem, inc=1,
                device_id=(p,), device_id_type=pl.DeviceIdType.MESH,
            )

        copies = []
        for c in range(2):
            cp = pltpu.make_async_copy(
                x_ref.at[pl.ds(c * m_half, m_half), :],
                buf_ref.at[c],
                copy_sems.at[c],
            )
            cp.start()
            copies.append(cp)

        copies[0].wait()
        comm_ref[my_pos] = jnp.max(buf_ref[0], axis=0, keepdims=True)
        copies[1].wait()
        comm_ref[my_pos] = jnp.maximum(
            comm_ref[my_pos], jnp.max(buf_ref[1], axis=0, keepdims=True)
        )

        pl.semaphore_wait(barrier_sem, N_DEV - 1)

        sends = []
        for k, p in enumerate(peers):
            rdma = pltpu.make_async_remote_copy(
                src_ref=comm_ref.at[my_pos],
                dst_ref=comm_ref.at[my_pos],
                send_sem=send_sems.at[k],
                recv_sem=recv_sems.at[my_pos],
                device_id=(p,),
                device_id_type=pl.DeviceIdType.MESH,
            )
            rdma.start()
            sends.append(rdma)

        acc = comm_ref[my_pos]
        for k, p in enumerate(peers):
            recv = pltpu.make_async_remote_copy(
                src_ref=comm_ref.at[p],
                dst_ref=comm_ref.at[p],
                send_sem=send_sems.at[k],
                recv_sem=recv_sems.at[p],
                device_id=(p,),
                device_id_type=pl.DeviceIdType.MESH,
            )
            recv.wait_recv()
            acc = jnp.maximum(acc, comm_ref[p])
        out_ref[:, :] = acc

        for rdma in sends:
            rdma.wait_send()

    return pl.pallas_call(
        body,
        out_shape=jax.ShapeDtypeStruct((1, n), x.dtype),
        in_specs=[pl.BlockSpec(memory_space=pl.ANY)],
        out_specs=pl.BlockSpec(memory_space=pltpu.VMEM),
        scratch_shapes=[
            pltpu.VMEM((2, m_per // 2, n), x.dtype),
            pltpu.VMEM((N_DEV, 1, n), x.dtype),
            pltpu.SemaphoreType.DMA((2,)),
            pltpu.SemaphoreType.DMA((N_DEV - 1,)),
            pltpu.SemaphoreType.DMA((N_DEV,)),
        ],
        compiler_params=pltpu.CompilerParams(collective_id=0),
    )(x)


# device time: 10523 ns/iter; 1.0525x vs baseline; 1.0525x over previous
import jax
import jax.numpy as jnp
from jax import lax
from jax.experimental import pallas as pl
from jax.experimental.pallas import tpu as pltpu

N_DEV = 8


def kernel(x):
    m_per, n = x.shape

    def body(x_ref, out_ref, comm_ref, send_sems, recv_sems):
        my_pos = lax.axis_index("i")
        peers = [(my_pos + off) % N_DEV for off in range(1, N_DEV)]

        barrier_sem = pltpu.get_barrier_semaphore()
        for p in peers:
            pl.semaphore_signal(
                barrier_sem, inc=1,
                device_id=(p,), device_id_type=pl.DeviceIdType.MESH,
            )

        comm_ref[my_pos] = jnp.max(x_ref[:, :], axis=0, keepdims=True)

        pl.semaphore_wait(barrier_sem, N_DEV - 1)

        sends = []
        for k, p in enumerate(peers):
            rdma = pltpu.make_async_remote_copy(
                src_ref=comm_ref.at[my_pos],
                dst_ref=comm_ref.at[my_pos],
                send_sem=send_sems.at[k],
                recv_sem=recv_sems.at[my_pos],
                device_id=(p,),
                device_id_type=pl.DeviceIdType.MESH,
            )
            rdma.start()
            sends.append(rdma)

        acc = comm_ref[my_pos]
        for k, p in enumerate(peers):
            recv = pltpu.make_async_remote_copy(
                src_ref=comm_ref.at[p],
                dst_ref=comm_ref.at[p],
                send_sem=send_sems.at[k],
                recv_sem=recv_sems.at[p],
                device_id=(p,),
                device_id_type=pl.DeviceIdType.MESH,
            )
            recv.wait_recv()
            acc = jnp.maximum(acc, comm_ref[p])
        out_ref[:, :] = acc

        for rdma in sends:
            rdma.wait_send()

    return pl.pallas_call(
        body,
        out_shape=jax.ShapeDtypeStruct((1, n), x.dtype),
        in_specs=[pl.BlockSpec(memory_space=pltpu.VMEM)],
        out_specs=pl.BlockSpec(memory_space=pltpu.VMEM),
        scratch_shapes=[
            pltpu.VMEM((N_DEV, 1, n), x.dtype),
            pltpu.SemaphoreType.DMA((N_DEV - 1,)),
            pltpu.SemaphoreType.DMA((N_DEV,)),
        ],
        compiler_params=pltpu.CompilerParams(collective_id=0),
    )(x)
